# baseline (speedup 1.0000x reference)
import functools

import jax
import jax.numpy as jnp
from jax import lax
from jax.experimental import pallas as pl
from jax.experimental.pallas import tpu as pltpu
from jax.experimental.pallas import tpu_sc as plsc

_TAPS = [(ic, dz, dy, dx) for dz in range(3) for dy in range(3)
         for dx in range(3) for ic in range(4)]
_KP = 112
_MP = 9728
_LW = 9984
_TBL = 941192
_CH = 2432
_NV = _CH // 16


def _tc_body(x0r, x1r, x2r, wr, br, q_ref, at_ref, *, zoff):
    z = pl.program_id(0) + zoff
    xr = [x0r, x1r, x2r]
    for k, (ic, dz, dy, dx) in enumerate(_TAPS):
        off = dy * 98 + dx
        at_ref[k, :] = xr[dz][0, ic, off:off + _MP]
    r = jnp.dot(wr[...], at_ref[...], preferred_element_type=jnp.float32)
    t = r + br[...][:, :1]
    iota_m = lax.broadcasted_iota(jnp.int32, (16, _MP), 1)
    gy = (iota_m // 98 + 1).astype(jnp.float32)
    gx = (iota_m % 98 + 1).astype(jnp.float32)
    zf = (z + 1).astype(jnp.float32)
    rowi = lax.broadcasted_iota(jnp.int32, (16, _MP), 0)
    grid = jnp.where(rowi < 3, zf, jnp.where(rowi < 6, gy, gx))
    q = grid + t
    q = jnp.minimum(jnp.maximum(q, 0.0), 96.0)
    q_ref[0] = q


def _tc_stage(xpf, wm, bias, nz, zoff):
    return pl.pallas_call(
        functools.partial(_tc_body, zoff=zoff),
        grid=(nz,),
        in_specs=[
            pl.BlockSpec((1, 4, _LW), lambda z: (z + zoff, 0, 0)),
            pl.BlockSpec((1, 4, _LW), lambda z: (z + zoff + 1, 0, 0)),
            pl.BlockSpec((1, 4, _LW), lambda z: (z + zoff + 2, 0, 0)),
            pl.BlockSpec((16, _KP), lambda z: (0, 0)),
            pl.BlockSpec((16, 128), lambda z: (0, 0)),
        ],
        out_specs=pl.BlockSpec((1, 16, _MP), lambda z: (z, 0, 0)),
        out_shape=jax.ShapeDtypeStruct((nz, 16, _MP), jnp.float32),
        scratch_shapes=[pltpu.VMEM((_KP, _MP), jnp.float32)],
    )(xpf, xpf, xpf, wm, bias)


def _sc_body(q_hbm, tab_hbm, out_hbm, qv, fv, iv8, gv, outv, sems, qsems,
             upt):
    info = plsc.get_sparse_core_info()
    nc = info.num_cores
    wid = lax.axis_index("s") * nc + lax.axis_index("c")
    nck = 4 * upt

    def coords(k):
        uid = wid * upt + k // 4
        z = uid // 3
        ch = uid - z * 3
        base = (k - (k // 4) * 4) * _CH
        return z, ch, base

    def prefetch(k, p):
        z, ch, base = coords(k)
        for c in range(3):
            pltpu.async_copy(q_hbm.at[z, c * 3 + ch, pl.ds(base, _CH)],
                             qv[p][c], qsems[p])

    def waitq(k, p):
        z, ch, base = coords(k)
        for c in range(3):
            pltpu.make_async_copy(q_hbm.at[z, c * 3 + ch, pl.ds(base, _CH)],
                                  qv[p][c], qsems[p]).wait()

    def idx_fire(k, p):
        z, ch, base = coords(k)
        chbase = ch * _TBL

        def idx_body(i, cc):
            sl = pl.ds(i * 16, 16)
            qz = qv[p][0][sl]
            qy = qv[p][1][sl]
            qx = qv[p][2][sl]
            az0 = qz * 9604.0
            az1 = (qz + 1.0) * 9604.0
            by0 = qy * 98.0
            by1 = (qy + 1.0) * 98.0
            corner = 0
            for az in (az0, az1):
                for by in (by0, by1):
                    for bx in (qx, qx + 1.0):
                        sm = (az + by) + bx
                        iv8[p][corner][sl] = sm.astype(jnp.int32) + chbase
                        corner += 1
            fv[p][0][sl] = qz.astype(jnp.int32).astype(jnp.float32) - qz
            fv[p][1][sl] = qy.astype(jnp.int32).astype(jnp.float32) - qy
            fv[p][2][sl] = qx.astype(jnp.int32).astype(jnp.float32) - qx
            return cc

        lax.fori_loop(0, _NV, idx_body, 0)

    def fire(k, p):
        for c8 in range(8):
            pltpu.async_copy(tab_hbm.at[iv8[p][c8]], gv[p][c8], sems[p])

    def finish(k, p):
        z, ch, base = coords(k)
        for c8 in range(8):
            pltpu.make_async_copy(tab_hbm.at[iv8[p][c8]], gv[p][c8],
                                  sems[p]).wait()

        def comb_body(i, cc):
            sl = pl.ds(i * 16, 16)
            fz = fv[p][0][sl]
            fy = fv[p][1][sl]
            fx = fv[p][2][sl]
            u0 = 1.0 + fz
            u1 = -fz
            v0 = 1.0 + fy
            v1 = -fy
            t0 = 1.0 + fx
            t1 = -fx
            g = [gv[p][c][sl] for c in range(8)]
            rz0 = v0 * (t0 * g[0] + t1 * g[1]) + v1 * (t0 * g[2] + t1 * g[3])
            rz1 = v0 * (t0 * g[4] + t1 * g[5]) + v1 * (t0 * g[6] + t1 * g[7])
            outv[p][sl] = u0 * rz0 + u1 * rz1
            return cc

        lax.fori_loop(0, _NV, comb_body, 0)
        pltpu.sync_copy(outv[p], out_hbm.at[ch, z, pl.ds(base, _CH)])

    prefetch(0, 0)
    waitq(0, 0)
    idx_fire(0, 0)
    prefetch(1, 1)
    fire(0, 0)

    def pipe_body(j, cc):
        k1 = 2 * j + 1
        k2 = 2 * j + 2
        waitq(k1, 1)
        idx_fire(k1, 1)
        prefetch(k2, 0)
        fire(k1, 1)
        finish(k1 - 1, 0)
        waitq(k2, 0)
        idx_fire(k2, 0)
        prefetch(jnp.minimum(k2 + 1, nck - 1), 1)
        fire(k2, 0)
        finish(k1, 1)
        return cc

    lax.fori_loop(0, (nck - 2) // 2, pipe_body, 0)
    waitq(nck - 1, 1)
    idx_fire(nck - 1, 1)
    fire(nck - 1, 1)
    finish(nck - 2, 0)
    finish(nck - 1, 1)


def _make_sc_stage(nz):
    upt = (nz * 3) // 32

    @functools.partial(
        pl.kernel,
        out_type=jax.ShapeDtypeStruct((3, nz, _MP), jnp.float32),
        mesh=plsc.VectorSubcoreMesh(core_axis_name="c", subcore_axis_name="s"),
        scratch_types=(
            [pltpu.VMEM((_CH,), jnp.float32)] * 6
            + [pltpu.VMEM((_CH,), jnp.float32)] * 6
            + [pltpu.VMEM((_CH,), jnp.int32)] * 16
            + [pltpu.VMEM((_CH,), jnp.float32)] * 16
            + [pltpu.VMEM((_CH,), jnp.float32)] * 2
            + [pltpu.SemaphoreType.DMA] * 4
        ),
    )
    def _sc_stage(q_hbm, tab_hbm, out_hbm, *scr):
        qv = (scr[0:3], scr[3:6])
        fv = (scr[6:9], scr[9:12])
        iv8 = (scr[12:20], scr[20:28])
        gv = (scr[28:36], scr[36:44])
        outv = scr[44:46]
        sems = scr[46:48]
        qsems = scr[48:50]
        _sc_body(q_hbm, tab_hbm, out_hbm, qv, fv, iv8, gv, outv, sems, qsems,
                 upt)

    return _sc_stage


_sc_stage_32 = _make_sc_stage(32)


def kernel(x, W_p, b_p):
    xpf = jnp.pad(x[0], ((0, 0), (1, 1), (1, 1), (1, 1))).reshape(4, 98, 9604)
    xpf = jnp.pad(xpf, ((0, 0), (0, 0), (0, _LW - 9604))).transpose(1, 0, 2)
    wm = jnp.stack([W_p[:, ic, dz, dy, dx] for (ic, dz, dy, dx) in _TAPS], axis=1)
    wm = jnp.pad(wm, ((0, 16 - 9), (0, _KP - 108)))
    bias = jnp.broadcast_to(jnp.pad(b_p, (0, 16 - 9))[:, None], (16, 128))

    tab = jnp.pad(x[0, :3], ((0, 0), (1, 1), (1, 1), (1, 1))).reshape(3 * _TBL)
    outs = []
    for zoff in (0, 32, 64):
        qh = _tc_stage(xpf, wm, bias, 32, zoff)
        outs.append(_sc_stage_32(qh, tab))
    out = jnp.concatenate(outs, axis=1)

    outm = out[:, :, :9408].reshape(3, 96, 96, 98)[..., :96][None]
    return jnp.concatenate([outm, x[:, 3:4]], axis=1)


# speedup vs baseline: 1.0485x; 1.0485x over previous
import functools

import jax
import jax.numpy as jnp
from jax import lax
from jax.experimental import pallas as pl
from jax.experimental.pallas import tpu as pltpu
from jax.experimental.pallas import tpu_sc as plsc

_TAPS = [(ic, dz, dy, dx) for dz in range(3) for dy in range(3)
         for dx in range(3) for ic in range(4)]
_KP = 112
_MP = 9728
_LW = 9984
_TBL = 941192
_CH = 2432
_NV = _CH // 16


def _tc_body(x0r, x1r, x2r, wr, br, q_ref, at_ref, *, zoff):
    z = pl.program_id(0) + zoff
    xr = [x0r, x1r, x2r]
    for k, (ic, dz, dy, dx) in enumerate(_TAPS):
        off = dy * 98 + dx
        at_ref[k, :] = xr[dz][0, ic, off:off + _MP]
    r = jnp.dot(wr[...], at_ref[...], preferred_element_type=jnp.float32)
    t = r + br[...][:, :1]
    iota_m = lax.broadcasted_iota(jnp.int32, (16, _MP), 1)
    gy = (iota_m // 98 + 1).astype(jnp.float32)
    gx = (iota_m % 98 + 1).astype(jnp.float32)
    zf = (z + 1).astype(jnp.float32)
    rowi = lax.broadcasted_iota(jnp.int32, (16, _MP), 0)
    grid = jnp.where(rowi < 3, zf, jnp.where(rowi < 6, gy, gx))
    q = grid + t
    q = jnp.minimum(jnp.maximum(q, 0.0), 96.0)
    q_ref[0] = q


def _tc_stage(xpf, wm, bias, nz, zoff):
    return pl.pallas_call(
        functools.partial(_tc_body, zoff=zoff),
        grid=(nz,),
        in_specs=[
            pl.BlockSpec((1, 4, _LW), lambda z: (z + zoff, 0, 0)),
            pl.BlockSpec((1, 4, _LW), lambda z: (z + zoff + 1, 0, 0)),
            pl.BlockSpec((1, 4, _LW), lambda z: (z + zoff + 2, 0, 0)),
            pl.BlockSpec((16, _KP), lambda z: (0, 0)),
            pl.BlockSpec((16, 128), lambda z: (0, 0)),
        ],
        out_specs=pl.BlockSpec((1, 16, _MP), lambda z: (z, 0, 0)),
        out_shape=jax.ShapeDtypeStruct((nz, 16, _MP), jnp.float32),
        scratch_shapes=[pltpu.VMEM((_KP, _MP), jnp.float32)],
    )(xpf, xpf, xpf, wm, bias)


def _sc_body(q_hbm, tab_hbm, out_hbm, qv, fv, iv8, gv, outv, sems, qsems,
             upt):
    info = plsc.get_sparse_core_info()
    nc = info.num_cores
    wid = lax.axis_index("s") * nc + lax.axis_index("c")
    nck = 4 * upt

    def coords(k):
        uid = wid * upt + k // 4
        z = uid // 3
        ch = uid - z * 3
        base = (k - (k // 4) * 4) * _CH
        return z, ch, base

    def prefetch(k, p):
        z, ch, base = coords(k)
        for c in range(3):
            pltpu.async_copy(q_hbm.at[z, c * 3 + ch, pl.ds(base, _CH)],
                             qv[p][c], qsems[p])

    def waitq(k, p):
        z, ch, base = coords(k)
        for c in range(3):
            pltpu.make_async_copy(q_hbm.at[z, c * 3 + ch, pl.ds(base, _CH)],
                                  qv[p][c], qsems[p]).wait()

    def idx_fire(k, p):
        z, ch, base = coords(k)
        chbase = ch * _TBL

        def idx_body(i, cc):
            sl = pl.ds(i * 16, 16)
            qz = qv[p][0][sl]
            qy = qv[p][1][sl]
            qx = qv[p][2][sl]
            az0 = qz * 9604.0
            az1 = (qz + 1.0) * 9604.0
            by0 = qy * 98.0
            by1 = (qy + 1.0) * 98.0
            corner = 0
            for az in (az0, az1):
                for by in (by0, by1):
                    for bx in (qx, qx + 1.0):
                        sm = (az + by) + bx
                        iv8[p][corner][sl] = sm.astype(jnp.int32) + chbase
                        corner += 1
            fv[p][0][sl] = qz.astype(jnp.int32).astype(jnp.float32) - qz
            fv[p][1][sl] = qy.astype(jnp.int32).astype(jnp.float32) - qy
            fv[p][2][sl] = qx.astype(jnp.int32).astype(jnp.float32) - qx
            return cc

        lax.fori_loop(0, _NV, idx_body, 0)

    def fire(k, p):
        for c8 in range(8):
            pltpu.async_copy(tab_hbm.at[iv8[p][c8]], gv[p][c8], sems[p])

    def finish(k, p):
        z, ch, base = coords(k)
        for c8 in range(8):
            pltpu.make_async_copy(tab_hbm.at[iv8[p][c8]], gv[p][c8],
                                  sems[p]).wait()

        def comb_body(i, cc):
            sl = pl.ds(i * 16, 16)
            fz = fv[p][0][sl]
            fy = fv[p][1][sl]
            fx = fv[p][2][sl]
            u0 = 1.0 + fz
            u1 = -fz
            v0 = 1.0 + fy
            v1 = -fy
            t0 = 1.0 + fx
            t1 = -fx
            g = [gv[p][c][sl] for c in range(8)]
            rz0 = v0 * (t0 * g[0] + t1 * g[1]) + v1 * (t0 * g[2] + t1 * g[3])
            rz1 = v0 * (t0 * g[4] + t1 * g[5]) + v1 * (t0 * g[6] + t1 * g[7])
            outv[p][sl] = u0 * rz0 + u1 * rz1
            return cc

        lax.fori_loop(0, _NV, comb_body, 0)
        pltpu.sync_copy(outv[p], out_hbm.at[ch, z, pl.ds(base, _CH)])

    prefetch(0, 0)
    waitq(0, 0)
    idx_fire(0, 0)
    prefetch(1, 1)
    fire(0, 0)

    def pipe_body(j, cc):
        k1 = 2 * j + 1
        k2 = 2 * j + 2
        waitq(k1, 1)
        idx_fire(k1, 1)
        prefetch(k2, 0)
        fire(k1, 1)
        finish(k1 - 1, 0)
        waitq(k2, 0)
        idx_fire(k2, 0)
        prefetch(jnp.minimum(k2 + 1, nck - 1), 1)
        fire(k2, 0)
        finish(k1, 1)
        return cc

    lax.fori_loop(0, (nck - 2) // 2, pipe_body, 0)
    waitq(nck - 1, 1)
    idx_fire(nck - 1, 1)
    fire(nck - 1, 1)
    finish(nck - 2, 0)
    finish(nck - 1, 1)


def _make_sc_stage(nz):
    upt = (nz * 3) // 32

    @functools.partial(
        pl.kernel,
        out_type=jax.ShapeDtypeStruct((3, nz, _MP), jnp.float32),
        mesh=plsc.VectorSubcoreMesh(core_axis_name="c", subcore_axis_name="s"),
        scratch_types=(
            [pltpu.VMEM((_CH,), jnp.float32)] * 6
            + [pltpu.VMEM((_CH,), jnp.float32)] * 6
            + [pltpu.VMEM((_CH,), jnp.int32)] * 16
            + [pltpu.VMEM((_CH,), jnp.float32)] * 16
            + [pltpu.VMEM((_CH,), jnp.float32)] * 2
            + [pltpu.SemaphoreType.DMA] * 4
        ),
    )
    def _sc_stage(q_hbm, tab_hbm, out_hbm, *scr):
        qv = (scr[0:3], scr[3:6])
        fv = (scr[6:9], scr[9:12])
        iv8 = (scr[12:20], scr[20:28])
        gv = (scr[28:36], scr[36:44])
        outv = scr[44:46]
        sems = scr[46:48]
        qsems = scr[48:50]
        _sc_body(q_hbm, tab_hbm, out_hbm, qv, fv, iv8, gv, outv, sems, qsems,
                 upt)

    return _sc_stage


_sc_stage_64 = _make_sc_stage(64)
_sc_stage_32 = _make_sc_stage(32)


def kernel(x, W_p, b_p):
    xpf = jnp.pad(x[0], ((0, 0), (1, 1), (1, 1), (1, 1))).reshape(4, 98, 9604)
    xpf = jnp.pad(xpf, ((0, 0), (0, 0), (0, _LW - 9604))).transpose(1, 0, 2)
    wm = jnp.stack([W_p[:, ic, dz, dy, dx] for (ic, dz, dy, dx) in _TAPS], axis=1)
    wm = jnp.pad(wm, ((0, 16 - 9), (0, _KP - 108)))
    bias = jnp.broadcast_to(jnp.pad(b_p, (0, 16 - 9))[:, None], (16, 128))

    tab = jnp.pad(x[0, :3], ((0, 0), (1, 1), (1, 1), (1, 1))).reshape(3 * _TBL)
    q1 = _tc_stage(xpf, wm, bias, 64, 0)
    o1 = _sc_stage_64(q1, tab)
    q2 = _tc_stage(xpf, wm, bias, 32, 64)
    o2 = _sc_stage_32(q2, tab)
    out = jnp.concatenate([o1, o2], axis=1)

    outm = out[:, :, :9408].reshape(3, 96, 96, 98)[..., :96][None]
    return jnp.concatenate([outm, x[:, 3:4]], axis=1)
